# baseline (speedup 1.0000x reference)
import functools

import jax
import jax.numpy as jnp
from jax import lax
from jax.experimental import pallas as pl
from jax.experimental.pallas import tpu as pltpu
from jax.experimental.pallas import tpu_sc as plsc

_E = 524288
_K = 20
_S = 2048
_B = 16
_C = 2
_NC = 2
_NS = 16
_NW = _NC * _NS
_EPW = _E // _NW
_ST = 21
_CH = 128
_CS = 137
_NCHUNK = _EPW // _CH
_GRP = _CH // 4

_P = (5.629329963841023e-07, 0.9999574661580921, -0.4992063824052593,
      0.3269723524219558, -0.22283471747775338, 0.13076335879445652,
      -0.052623955162732786, 0.01011890169509671)


def _log1p_poly(z):
    acc = jnp.full((16,), _P[7], jnp.float32)
    for c in (_P[6], _P[5], _P[4], _P[3], _P[2], _P[1], _P[0]):
        acc = acc * z + c
    return acc


def _sc_kernel(th_hbm, al_hbm, lb_hbm, sg_hbm,
               th0_hbm, al0_hbm, lb0_hbm, sg0_hbm,
               adj_out, alp_out, cnt_out, adj0_out, alp0_out, cnt0_out,
               th_v0, th_v1, al_v0, al_v1, lb_v0, lb_v1, sg_v0, sg_v1,
               adj_acc, alp_acc, cnt_acc, row_buf, sem0, sem1):
    th_vs, al_vs = (th_v0, th_v1), (al_v0, al_v1)
    lb_vs, sg_vs = (lb_v0, lb_v1), (sg_v0, sg_v1)
    wid = lax.axis_index("s") * _NC + lax.axis_index("c")

    iota = lax.iota(jnp.int32, 16)
    kvs = []
    evs = []
    for j in range(5):
        fl = iota + 16 * j
        kvs.append(lax.rem(fl, _K))
        ev = ((fl >= _K).astype(jnp.int32) + (fl >= 2 * _K).astype(jnp.int32)
              + (fl >= 3 * _K).astype(jnp.int32))
        evs.append(ev)
    ones = jnp.full((16,), 1.0, jnp.float32)
    sixteenth = jnp.full((16,), 1.0 / 16.0, jnp.float32)
    eighth = jnp.full((16,), 1.0 / 8.0, jnp.float32)
    zeros = jnp.zeros((16,), jnp.float32)

    m4 = iota < 4
    pidx = [[evs[j] + 4 * q for j in range(5)] for q in range(4)]
    cidx = [jnp.where(m4, iota, 3) + 4 * q for q in range(4)]
    i25 = iota * _ST

    def run_branch(th, al, lb, sg, a_out, l_out, c_out):
        @plsc.parallel_loop(0, _ST * _S // 16)
        def zero_kS(i):
            adj_acc[pl.ds(i * 16, 16)] = zeros
            alp_acc[pl.ds(i * 16, 16)] = zeros

        @plsc.parallel_loop(0, _S // 16)
        def zero_S(i):
            cnt_acc[pl.ds(i * 16, 16)] = zeros

        sems = (sem0, sem1)

        def start(ci, b):
            eoff = wid * _EPW + ci * _CH
            pltpu.async_copy(th.at[:, pl.ds(eoff, _CH)],
                             th_vs[b].at[:, pl.ds(0, _CH)], sems[b])
            pltpu.async_copy(al.at[:, pl.ds(eoff, _CH)],
                             al_vs[b].at[:, pl.ds(0, _CH)], sems[b])
            pltpu.async_copy(lb.at[pl.ds(eoff, _CH)], lb_vs[b], sems[b])
            pltpu.async_copy(sg.at[pl.ds(eoff, _CH)], sg_vs[b], sems[b])

        def wait_slot(b):
            pltpu.make_async_copy(th.at[:, pl.ds(0, _CH)],
                                  th_vs[b].at[:, pl.ds(0, _CH)],
                                  sems[b]).wait()
            pltpu.make_async_copy(al.at[:, pl.ds(0, _CH)],
                                  al_vs[b].at[:, pl.ds(0, _CH)],
                                  sems[b]).wait()
            pltpu.make_async_copy(lb.at[pl.ds(0, _CH)], lb_vs[b],
                                  sems[b]).wait()
            pltpu.make_async_copy(sg.at[pl.ds(0, _CH)], sg_vs[b],
                                  sems[b]).wait()

        def compute(b):
            @plsc.parallel_loop(0, _CH // 16, unroll=2)
            def block_body(blk):
                eb16 = blk * 16
                s16 = sg_vs[b][pl.ds(eb16, 16)]
                y16 = lb_vs[b][pl.ds(eb16, 16)]
                st16 = s16 * _ST
                for q in range(4):
                    for j in range(5):
                        pid = pidx[q][j]
                        ev = eb16 + pid
                        tgt = jnp.take_along_axis(st16, pid, 0) + kvs[j]
                        y = jnp.take_along_axis(y16, pid, 0)
                        t = plsc.load_gather(th_vs[b], [kvs[j], ev])
                        z = jnp.exp(-jnp.abs(t))
                        loss = jnp.maximum(t, 0.0) - t * y + _log1p_poly(z)
                        plsc.addupdate_scatter(adj_acc, [tgt], loss)
                        av = plsc.load_gather(al_vs[b], [kvs[j], ev])
                        plsc.addupdate_scatter(alp_acc, [tgt], av)
                    sq = jnp.take_along_axis(s16, cidx[q], 0)
                    plsc.addupdate_scatter(cnt_acc, [sq], ones, mask=m4)

        start(0, 0)
        start(1, 1)

        def chunk_pair(i, carry):
            ci = i * 2
            for b in range(2):
                wait_slot(b)
                compute(b)

                @pl.when(ci + 2 + b < _NCHUNK)
                def _():
                    start(ci + 2 + b, b)
            return carry
        lax.fori_loop(0, _NCHUNK // 2, chunk_pair, 0)

        def flush(acc, out2d):
            def row(k, carry):
                @plsc.parallel_loop(0, _S // 16)
                def blk(sb):
                    v = plsc.load_gather(acc, [i25 + (sb * 16 * _ST + k)])
                    row_buf[pl.ds(sb * 16, 16)] = v
                pltpu.sync_copy(row_buf, out2d.at[wid, pl.ds(k * _S, _S)])
                return carry
            lax.fori_loop(0, _K, row, 0)
        flush(adj_acc, a_out)
        flush(alp_acc, l_out)
        pltpu.sync_copy(cnt_acc, c_out.at[wid])

    run_branch(th_hbm, al_hbm, lb_hbm, sg_hbm, adj_out, alp_out, cnt_out)
    run_branch(th0_hbm, al0_hbm, lb0_hbm, sg0_hbm, adj0_out, alp0_out, cnt0_out)


_sc_call = functools.partial(
    pl.kernel,
    out_type=[
        jax.ShapeDtypeStruct((_NW, _K * _S), jnp.float32),
        jax.ShapeDtypeStruct((_NW, _K * _S), jnp.float32),
        jax.ShapeDtypeStruct((_NW, _S), jnp.float32),
        jax.ShapeDtypeStruct((_NW, _K * _S), jnp.float32),
        jax.ShapeDtypeStruct((_NW, _K * _S), jnp.float32),
        jax.ShapeDtypeStruct((_NW, _S), jnp.float32),
    ],
    mesh=plsc.VectorSubcoreMesh(core_axis_name="c", subcore_axis_name="s",
                                num_cores=_NC, num_subcores=_NS),
    scratch_types=[
        pltpu.VMEM((_K, _CS), jnp.float32),
        pltpu.VMEM((_K, _CS), jnp.float32),
        pltpu.VMEM((_K, _CS), jnp.float32),
        pltpu.VMEM((_K, _CS), jnp.float32),
        pltpu.VMEM((_CH,), jnp.float32),
        pltpu.VMEM((_CH,), jnp.float32),
        pltpu.VMEM((_CH,), jnp.int32),
        pltpu.VMEM((_CH,), jnp.int32),
        pltpu.VMEM((_ST * _S,), jnp.float32),
        pltpu.VMEM((_ST * _S,), jnp.float32),
        pltpu.VMEM((_S,), jnp.float32),
        pltpu.VMEM((_S,), jnp.float32),
        pltpu.SemaphoreType.DMA,
        pltpu.SemaphoreType.DMA,
    ],
    compiler_params=pltpu.CompilerParams(needs_layout_passes=False),
)(_sc_kernel)


def _lse0(x):
    m = jnp.max(x, axis=0, keepdims=True)
    return jnp.log(jnp.sum(jnp.exp(x - m), axis=0, keepdims=True)) + m


def _tc_finish_kernel(adj_ref, alp_ref, cnt_ref, adj0_ref, alp0_ref, cnt0_ref,
                      ns_ref, ns0_ref, out_ref):
    f32 = jnp.float32
    ri = lax.broadcasted_iota(jnp.int32, (_S, _B * _C), 0)
    cj = lax.broadcasted_iota(jnp.int32, (_S, _B * _C), 1)
    bucket = (ri // (_S // (_B * _C)) == cj).astype(f32)
    pi = lax.broadcasted_iota(jnp.int32, (_B * _C, _B), 0)
    pj = lax.broadcasted_iota(jnp.int32, (_B * _C, _B), 1)
    even = (pi == 2 * pj).astype(f32)
    odd = (pi == 2 * pj + 1).astype(f32)

    cnt = jnp.sum(cnt_ref[...], axis=0, keepdims=True)
    cnt0 = jnp.sum(cnt0_ref[...], axis=0, keepdims=True)

    def branch(adj_p, alp_p, cnt_norm, cnt_bc):
        adj = jnp.sum(adj_p.reshape(_NW, _K, _S), axis=0)
        alp = jnp.sum(alp_p.reshape(_NW, _K, _S), axis=0) / cnt_norm
        lsm = alp - _lse0(alp)
        log_prob = _lse0(lsm - adj)
        bc_lp = jnp.dot(log_prob, bucket, precision=lax.Precision.HIGHEST)
        bc_cn = jnp.dot(cnt_bc, bucket, precision=lax.Precision.HIGHEST)
        bc_loss = bc_lp / bc_cn
        e = jnp.dot(bc_loss, even, precision=lax.Precision.HIGHEST)
        o = jnp.dot(bc_loss, odd, precision=lax.Precision.HIGHEST)
        m = jnp.maximum(e, o)
        b_loss = -(jnp.log(jnp.exp(e - m) + jnp.exp(o - m)) + m)
        return jnp.sum(b_loss) / _B

    loss = branch(adj_ref[...], alp_ref[...], cnt, cnt)
    loss0 = branch(adj0_ref[...], alp0_ref[...], cnt, cnt0)

    d = ns_ref[...] - ns0_ref[...]
    trace_d = jnp.sum(jnp.sqrt(jnp.sum(d * d, axis=1)))

    out_ref[0, 0] = 10.0 * (loss + loss0) + trace_d


def kernel(label, label0, log_theta, log_theta0, log_alpha, log_alpha0,
           node_state, node_state0, subgraph_idx, subgraph_idx0,
           subgraph_idx_base, subgraph_idx_base0, num_canonical_order):
    adj, alp, cnt, adj0, alp0, cnt0 = _sc_call(
        log_theta.T, log_alpha.T, label, subgraph_idx,
        log_theta0.T, log_alpha0.T, label0, subgraph_idx0)

    out = pl.pallas_call(
        _tc_finish_kernel,
        out_shape=jax.ShapeDtypeStruct((1, 1), jnp.float32),
        in_specs=[pl.BlockSpec(memory_space=pltpu.VMEM)] * 8,
        out_specs=pl.BlockSpec(memory_space=pltpu.SMEM),
    )(adj.reshape(_NW * _K, _S), alp.reshape(_NW * _K, _S), cnt,
      adj0.reshape(_NW * _K, _S), alp0.reshape(_NW * _K, _S), cnt0,
      node_state, node_state0)
    return out[0, 0]


# speedup vs baseline: 1.9457x; 1.9457x over previous
import functools

import jax
import jax.numpy as jnp
from jax import lax
from jax.experimental import pallas as pl
from jax.experimental.pallas import tpu as pltpu
from jax.experimental.pallas import tpu_sc as plsc

_E = 524288
_K = 20
_S = 2048
_B = 16
_C = 2
_NC = 2
_NS = 16
_NW = _NC * _NS
_EPW = _E // _NW
_ST = 21
_CH = 128
_CS = 137
_NCHUNK = _EPW // _CH
_GRP = _CH // 4

_P = (3.511021356650268e-06, 0.9997923620654495, -0.49697743071907685,
      0.31458917398920905, -0.1887808235491981, 0.08172564529133709,
      -0.01720779923058697)


def _log1p_poly(z):
    acc = jnp.full((16,), _P[6], jnp.float32)
    for c in (_P[5], _P[4], _P[3], _P[2], _P[1], _P[0]):
        acc = acc * z + c
    return acc


def _sc_kernel(th_hbm, al_hbm, lb_hbm, sg_hbm,
               th0_hbm, al0_hbm, lb0_hbm, sg0_hbm,
               adj_out, alp_out, cnt_out, adj0_out, alp0_out, cnt0_out,
               th_v0, th_v1, al_v0, al_v1, lb_v0, lb_v1, sg_v0, sg_v1,
               adj_acc, alp_acc, cnt_acc, row_buf, sem0, sem1):
    th_vs, al_vs = (th_v0, th_v1), (al_v0, al_v1)
    lb_vs, sg_vs = (lb_v0, lb_v1), (sg_v0, sg_v1)
    wid = lax.axis_index("s") * _NC + lax.axis_index("c")

    iota = lax.iota(jnp.int32, 16)
    kvs = []
    evs = []
    for j in range(5):
        fl = iota + 16 * j
        kvs.append(lax.rem(fl, _K))
        ev = ((fl >= _K).astype(jnp.int32) + (fl >= 2 * _K).astype(jnp.int32)
              + (fl >= 3 * _K).astype(jnp.int32))
        evs.append(ev)
    ones = jnp.full((16,), 1.0, jnp.float32)
    sixteenth = jnp.full((16,), 1.0 / 16.0, jnp.float32)
    eighth = jnp.full((16,), 1.0 / 8.0, jnp.float32)
    zeros = jnp.zeros((16,), jnp.float32)

    cmask = [kvs[j] == 0 for j in range(4)]
    i25 = iota * _ST

    def run_branch(th, al, lb, sg, a_out, l_out, c_out):
        @plsc.parallel_loop(0, _ST * _S // 16)
        def zero_kS(i):
            adj_acc[pl.ds(i * 16, 16)] = zeros
            alp_acc[pl.ds(i * 16, 16)] = zeros

        @plsc.parallel_loop(0, _S // 16)
        def zero_S(i):
            cnt_acc[pl.ds(i * 16, 16)] = zeros

        sems = (sem0, sem1)

        def start(ci, b):
            eoff = wid * _EPW + ci * _CH
            pltpu.async_copy(th.at[:, pl.ds(eoff, _CH)],
                             th_vs[b].at[:, pl.ds(0, _CH)], sems[b])
            pltpu.async_copy(al.at[:, pl.ds(eoff, _CH)],
                             al_vs[b].at[:, pl.ds(0, _CH)], sems[b])
            pltpu.async_copy(lb.at[pl.ds(eoff, _CH)], lb_vs[b], sems[b])
            pltpu.async_copy(sg.at[pl.ds(eoff, _CH)], sg_vs[b], sems[b])

        def wait_slot(b):
            pltpu.make_async_copy(th.at[:, pl.ds(0, _CH)],
                                  th_vs[b].at[:, pl.ds(0, _CH)],
                                  sems[b]).wait()
            pltpu.make_async_copy(al.at[:, pl.ds(0, _CH)],
                                  al_vs[b].at[:, pl.ds(0, _CH)],
                                  sems[b]).wait()
            pltpu.make_async_copy(lb.at[pl.ds(0, _CH)], lb_vs[b],
                                  sems[b]).wait()
            pltpu.make_async_copy(sg.at[pl.ds(0, _CH)], sg_vs[b],
                                  sems[b]).wait()

        def compute(b):
            @plsc.parallel_loop(0, _GRP, unroll=4)
            def group_body(g):
                eb = g * 4
                for j in range(5):
                    ev = evs[j] + eb
                    seg = plsc.load_gather(sg_vs[b], [ev])
                    y = plsc.load_gather(lb_vs[b], [ev])
                    tgt = seg * _ST + kvs[j]
                    t = plsc.load_gather(th_vs[b], [kvs[j], ev])
                    z = jnp.exp(-jnp.abs(t))
                    loss = jnp.maximum(t, 0.0) - t * y + _log1p_poly(z)
                    plsc.addupdate_scatter(adj_acc, [tgt], loss)
                    av = plsc.load_gather(al_vs[b], [kvs[j], ev])
                    plsc.addupdate_scatter(alp_acc, [tgt], av)
                    if j < 4:
                        plsc.addupdate_scatter(cnt_acc, [seg], ones,
                                               mask=cmask[j])

        start(0, 0)
        start(1, 1)

        def chunk_pair(i, carry):
            ci = i * 2
            for b in range(2):
                wait_slot(b)
                compute(b)

                @pl.when(ci + 2 + b < _NCHUNK)
                def _():
                    start(ci + 2 + b, b)
            return carry
        lax.fori_loop(0, _NCHUNK // 2, chunk_pair, 0)

        def flush(acc, out2d):
            def row(k, carry):
                @plsc.parallel_loop(0, _S // 16)
                def blk(sb):
                    v = plsc.load_gather(acc, [i25 + (sb * 16 * _ST + k)])
                    row_buf[pl.ds(sb * 16, 16)] = v
                pltpu.sync_copy(row_buf, out2d.at[wid, pl.ds(k * _S, _S)])
                return carry
            lax.fori_loop(0, _K, row, 0)
        flush(adj_acc, a_out)
        flush(alp_acc, l_out)
        pltpu.sync_copy(cnt_acc, c_out.at[wid])

    run_branch(th_hbm, al_hbm, lb_hbm, sg_hbm, adj_out, alp_out, cnt_out)
    run_branch(th0_hbm, al0_hbm, lb0_hbm, sg0_hbm, adj0_out, alp0_out, cnt0_out)


_sc_call = functools.partial(
    pl.kernel,
    out_type=[
        jax.ShapeDtypeStruct((_NW, _K * _S), jnp.float32),
        jax.ShapeDtypeStruct((_NW, _K * _S), jnp.float32),
        jax.ShapeDtypeStruct((_NW, _S), jnp.float32),
        jax.ShapeDtypeStruct((_NW, _K * _S), jnp.float32),
        jax.ShapeDtypeStruct((_NW, _K * _S), jnp.float32),
        jax.ShapeDtypeStruct((_NW, _S), jnp.float32),
    ],
    mesh=plsc.VectorSubcoreMesh(core_axis_name="c", subcore_axis_name="s",
                                num_cores=_NC, num_subcores=_NS),
    scratch_types=[
        pltpu.VMEM((_K, _CS), jnp.float32),
        pltpu.VMEM((_K, _CS), jnp.float32),
        pltpu.VMEM((_K, _CS), jnp.float32),
        pltpu.VMEM((_K, _CS), jnp.float32),
        pltpu.VMEM((_CH,), jnp.float32),
        pltpu.VMEM((_CH,), jnp.float32),
        pltpu.VMEM((_CH,), jnp.int32),
        pltpu.VMEM((_CH,), jnp.int32),
        pltpu.VMEM((_ST * _S,), jnp.float32),
        pltpu.VMEM((_ST * _S,), jnp.float32),
        pltpu.VMEM((_S,), jnp.float32),
        pltpu.VMEM((_S,), jnp.float32),
        pltpu.SemaphoreType.DMA,
        pltpu.SemaphoreType.DMA,
    ],
    compiler_params=pltpu.CompilerParams(needs_layout_passes=False),
)(_sc_kernel)


def _lse0(x):
    m = jnp.max(x, axis=0, keepdims=True)
    return jnp.log(jnp.sum(jnp.exp(x - m), axis=0, keepdims=True)) + m


def _tc_finish_kernel(adj_ref, alp_ref, cnt_ref, adj0_ref, alp0_ref, cnt0_ref,
                      ns_ref, ns0_ref, out_ref):
    f32 = jnp.float32
    ri = lax.broadcasted_iota(jnp.int32, (_S, _B * _C), 0)
    cj = lax.broadcasted_iota(jnp.int32, (_S, _B * _C), 1)
    bucket = (ri // (_S // (_B * _C)) == cj).astype(f32)
    pi = lax.broadcasted_iota(jnp.int32, (_B * _C, _B), 0)
    pj = lax.broadcasted_iota(jnp.int32, (_B * _C, _B), 1)
    even = (pi == 2 * pj).astype(f32)
    odd = (pi == 2 * pj + 1).astype(f32)

    cnt = jnp.sum(cnt_ref[...], axis=0, keepdims=True)
    cnt0 = jnp.sum(cnt0_ref[...], axis=0, keepdims=True)

    def branch(adj_p, alp_p, cnt_norm, cnt_bc):
        adj = jnp.sum(adj_p.reshape(_NW, _K, _S), axis=0)
        alp = jnp.sum(alp_p.reshape(_NW, _K, _S), axis=0) / cnt_norm
        lsm = alp - _lse0(alp)
        log_prob = _lse0(lsm - adj)
        bc_lp = jnp.dot(log_prob, bucket, precision=lax.Precision.HIGHEST)
        bc_cn = jnp.dot(cnt_bc, bucket, precision=lax.Precision.HIGHEST)
        bc_loss = bc_lp / bc_cn
        e = jnp.dot(bc_loss, even, precision=lax.Precision.HIGHEST)
        o = jnp.dot(bc_loss, odd, precision=lax.Precision.HIGHEST)
        m = jnp.maximum(e, o)
        b_loss = -(jnp.log(jnp.exp(e - m) + jnp.exp(o - m)) + m)
        return jnp.sum(b_loss) / _B

    loss = branch(adj_ref[...], alp_ref[...], cnt, cnt)
    loss0 = branch(adj0_ref[...], alp0_ref[...], cnt, cnt0)

    d = ns_ref[...] - ns0_ref[...]
    trace_d = jnp.sum(jnp.sqrt(jnp.sum(d * d, axis=1)))

    out_ref[0, 0] = 10.0 * (loss + loss0) + trace_d


def kernel(label, label0, log_theta, log_theta0, log_alpha, log_alpha0,
           node_state, node_state0, subgraph_idx, subgraph_idx0,
           subgraph_idx_base, subgraph_idx_base0, num_canonical_order):
    adj, alp, cnt, adj0, alp0, cnt0 = _sc_call(
        log_theta.T, log_alpha.T, label, subgraph_idx,
        log_theta0.T, log_alpha0.T, label0, subgraph_idx0)

    out = pl.pallas_call(
        _tc_finish_kernel,
        out_shape=jax.ShapeDtypeStruct((1, 1), jnp.float32),
        in_specs=[pl.BlockSpec(memory_space=pltpu.VMEM)] * 8,
        out_specs=pl.BlockSpec(memory_space=pltpu.SMEM),
    )(adj.reshape(_NW * _K, _S), alp.reshape(_NW * _K, _S), cnt,
      adj0.reshape(_NW * _K, _S), alp0.reshape(_NW * _K, _S), cnt0,
      node_state, node_state0)
    return out[0, 0]


# speedup vs baseline: 2.1052x; 1.0820x over previous
import functools

import jax
import jax.numpy as jnp
from jax import lax
from jax.experimental import pallas as pl
from jax.experimental.pallas import tpu as pltpu
from jax.experimental.pallas import tpu_sc as plsc

_E = 524288
_K = 20
_S = 2048
_B = 16
_C = 2
_NC = 2
_NS = 16
_NW = _NC * _NS
_EPW = _E // _NW
_ST = 21
_CH = 128
_CS = 137
_NCHUNK = _EPW // _CH
_GRP = _CH // 4

_P = (3.511021356650268e-06, 0.9997923620654495, -0.49697743071907685,
      0.31458917398920905, -0.1887808235491981, 0.08172564529133709,
      -0.01720779923058697)


def _log1p_poly(z):
    acc = jnp.full((16,), _P[6], jnp.float32)
    for c in (_P[5], _P[4], _P[3], _P[2], _P[1], _P[0]):
        acc = acc * z + c
    return acc


def _sc_kernel(th_hbm, al_hbm, lb_hbm, sg_hbm,
               th0_hbm, al0_hbm, lb0_hbm, sg0_hbm,
               adj_out, alp_out, cnt_out, adj0_out, alp0_out, cnt0_out,
               th_v0, th_v1, al_v0, al_v1, lb_v0, lb_v1, sg_v0, sg_v1,
               adj_acc, alp_acc, cnt_acc, row_buf, sem0, sem1):
    th_vs, al_vs = (th_v0, th_v1), (al_v0, al_v1)
    lb_vs, sg_vs = (lb_v0, lb_v1), (sg_v0, sg_v1)
    wid = lax.axis_index("s") * _NC + lax.axis_index("c")

    iota = lax.iota(jnp.int32, 16)
    kvs = []
    evs = []
    for j in range(5):
        fl = iota + 16 * j
        kvs.append(lax.rem(fl, _K))
        ev = ((fl >= _K).astype(jnp.int32) + (fl >= 2 * _K).astype(jnp.int32)
              + (fl >= 3 * _K).astype(jnp.int32))
        evs.append(ev)
    ones = jnp.full((16,), 1.0, jnp.float32)
    sixteenth = jnp.full((16,), 1.0 / 16.0, jnp.float32)
    eighth = jnp.full((16,), 1.0 / 8.0, jnp.float32)
    zeros = jnp.zeros((16,), jnp.float32)

    cmask = [kvs[j] == 0 for j in range(4)]
    i25 = iota * _ST

    def run_branch(th, al, lb, sg, a_out, l_out, c_out):
        @plsc.parallel_loop(0, _ST * _S // 16)
        def zero_kS(i):
            adj_acc[pl.ds(i * 16, 16)] = zeros
            alp_acc[pl.ds(i * 16, 16)] = zeros

        @plsc.parallel_loop(0, _S // 16)
        def zero_S(i):
            cnt_acc[pl.ds(i * 16, 16)] = zeros

        sems = (sem0, sem1)

        def start(ci, b):
            eoff = wid * _EPW + ci * _CH
            pltpu.async_copy(th.at[:, pl.ds(eoff, _CH)],
                             th_vs[b].at[:, pl.ds(0, _CH)], sems[b])
            pltpu.async_copy(al.at[:, pl.ds(eoff, _CH)],
                             al_vs[b].at[:, pl.ds(0, _CH)], sems[b])
            pltpu.async_copy(lb.at[pl.ds(eoff, _CH)], lb_vs[b], sems[b])
            pltpu.async_copy(sg.at[pl.ds(eoff, _CH)], sg_vs[b], sems[b])

        def wait_slot(b):
            pltpu.make_async_copy(th.at[:, pl.ds(0, _CH)],
                                  th_vs[b].at[:, pl.ds(0, _CH)],
                                  sems[b]).wait()
            pltpu.make_async_copy(al.at[:, pl.ds(0, _CH)],
                                  al_vs[b].at[:, pl.ds(0, _CH)],
                                  sems[b]).wait()
            pltpu.make_async_copy(lb.at[pl.ds(0, _CH)], lb_vs[b],
                                  sems[b]).wait()
            pltpu.make_async_copy(sg.at[pl.ds(0, _CH)], sg_vs[b],
                                  sems[b]).wait()

        def compute(b):
            @plsc.parallel_loop(0, _GRP, unroll=2)
            def group_body(g):
                eb = g * 4
                for j in range(5):
                    ev = evs[j] + eb
                    seg = plsc.load_gather(sg_vs[b], [ev])
                    y = plsc.load_gather(lb_vs[b], [ev])
                    tgt = seg * _ST + kvs[j]
                    t = plsc.load_gather(th_vs[b], [kvs[j], ev])
                    z = jnp.exp(-jnp.abs(t))
                    loss = jnp.maximum(t, 0.0) - t * y + _log1p_poly(z)
                    plsc.addupdate_scatter(adj_acc, [tgt], loss)
                    av = plsc.load_gather(al_vs[b], [kvs[j], ev])
                    plsc.addupdate_scatter(alp_acc, [tgt], av)
                    if j < 4:
                        plsc.addupdate_scatter(cnt_acc, [seg], ones,
                                               mask=cmask[j])

        start(0, 0)
        start(1, 1)

        def chunk_pair(i, carry):
            ci = i * 2
            for b in range(2):
                wait_slot(b)
                compute(b)

                @pl.when(ci + 2 + b < _NCHUNK)
                def _():
                    start(ci + 2 + b, b)
            return carry
        lax.fori_loop(0, _NCHUNK // 2, chunk_pair, 0)

        def flush(acc, out2d):
            def row(k, carry):
                @plsc.parallel_loop(0, _S // 16)
                def blk(sb):
                    v = plsc.load_gather(acc, [i25 + (sb * 16 * _ST + k)])
                    row_buf[pl.ds(sb * 16, 16)] = v
                pltpu.sync_copy(row_buf, out2d.at[wid, pl.ds(k * _S, _S)])
                return carry
            lax.fori_loop(0, _K, row, 0)
        flush(adj_acc, a_out)
        flush(alp_acc, l_out)
        pltpu.sync_copy(cnt_acc, c_out.at[wid])

    run_branch(th_hbm, al_hbm, lb_hbm, sg_hbm, adj_out, alp_out, cnt_out)
    run_branch(th0_hbm, al0_hbm, lb0_hbm, sg0_hbm, adj0_out, alp0_out, cnt0_out)


_sc_call = functools.partial(
    pl.kernel,
    out_type=[
        jax.ShapeDtypeStruct((_NW, _K * _S), jnp.float32),
        jax.ShapeDtypeStruct((_NW, _K * _S), jnp.float32),
        jax.ShapeDtypeStruct((_NW, _S), jnp.float32),
        jax.ShapeDtypeStruct((_NW, _K * _S), jnp.float32),
        jax.ShapeDtypeStruct((_NW, _K * _S), jnp.float32),
        jax.ShapeDtypeStruct((_NW, _S), jnp.float32),
    ],
    mesh=plsc.VectorSubcoreMesh(core_axis_name="c", subcore_axis_name="s",
                                num_cores=_NC, num_subcores=_NS),
    scratch_types=[
        pltpu.VMEM((_K, _CS), jnp.float32),
        pltpu.VMEM((_K, _CS), jnp.float32),
        pltpu.VMEM((_K, _CS), jnp.float32),
        pltpu.VMEM((_K, _CS), jnp.float32),
        pltpu.VMEM((_CH,), jnp.float32),
        pltpu.VMEM((_CH,), jnp.float32),
        pltpu.VMEM((_CH,), jnp.int32),
        pltpu.VMEM((_CH,), jnp.int32),
        pltpu.VMEM((_ST * _S,), jnp.float32),
        pltpu.VMEM((_ST * _S,), jnp.float32),
        pltpu.VMEM((_S,), jnp.float32),
        pltpu.VMEM((_S,), jnp.float32),
        pltpu.SemaphoreType.DMA,
        pltpu.SemaphoreType.DMA,
    ],
    compiler_params=pltpu.CompilerParams(needs_layout_passes=False),
)(_sc_kernel)


def _lse0(x):
    m = jnp.max(x, axis=0, keepdims=True)
    return jnp.log(jnp.sum(jnp.exp(x - m), axis=0, keepdims=True)) + m


def _tc_finish_kernel(adj_ref, alp_ref, cnt_ref, adj0_ref, alp0_ref, cnt0_ref,
                      ns_ref, ns0_ref, out_ref):
    f32 = jnp.float32
    ri = lax.broadcasted_iota(jnp.int32, (_S, _B * _C), 0)
    cj = lax.broadcasted_iota(jnp.int32, (_S, _B * _C), 1)
    bucket = (ri // (_S // (_B * _C)) == cj).astype(f32)
    pi = lax.broadcasted_iota(jnp.int32, (_B * _C, _B), 0)
    pj = lax.broadcasted_iota(jnp.int32, (_B * _C, _B), 1)
    even = (pi == 2 * pj).astype(f32)
    odd = (pi == 2 * pj + 1).astype(f32)

    cnt = jnp.sum(cnt_ref[...], axis=0, keepdims=True)
    cnt0 = jnp.sum(cnt0_ref[...], axis=0, keepdims=True)

    def branch(adj_p, alp_p, cnt_norm, cnt_bc):
        adj = jnp.sum(adj_p.reshape(_NW, _K, _S), axis=0)
        alp = jnp.sum(alp_p.reshape(_NW, _K, _S), axis=0) / cnt_norm
        lsm = alp - _lse0(alp)
        log_prob = _lse0(lsm - adj)
        bc_lp = jnp.dot(log_prob, bucket, precision=lax.Precision.HIGHEST)
        bc_cn = jnp.dot(cnt_bc, bucket, precision=lax.Precision.HIGHEST)
        bc_loss = bc_lp / bc_cn
        e = jnp.dot(bc_loss, even, precision=lax.Precision.HIGHEST)
        o = jnp.dot(bc_loss, odd, precision=lax.Precision.HIGHEST)
        m = jnp.maximum(e, o)
        b_loss = -(jnp.log(jnp.exp(e - m) + jnp.exp(o - m)) + m)
        return jnp.sum(b_loss) / _B

    loss = branch(adj_ref[...], alp_ref[...], cnt, cnt)
    loss0 = branch(adj0_ref[...], alp0_ref[...], cnt, cnt0)

    d = ns_ref[...] - ns0_ref[...]
    trace_d = jnp.sum(jnp.sqrt(jnp.sum(d * d, axis=1)))

    out_ref[0, 0] = 10.0 * (loss + loss0) + trace_d


def kernel(label, label0, log_theta, log_theta0, log_alpha, log_alpha0,
           node_state, node_state0, subgraph_idx, subgraph_idx0,
           subgraph_idx_base, subgraph_idx_base0, num_canonical_order):
    adj, alp, cnt, adj0, alp0, cnt0 = _sc_call(
        log_theta.T, log_alpha.T, label, subgraph_idx,
        log_theta0.T, log_alpha0.T, label0, subgraph_idx0)

    out = pl.pallas_call(
        _tc_finish_kernel,
        out_shape=jax.ShapeDtypeStruct((1, 1), jnp.float32),
        in_specs=[pl.BlockSpec(memory_space=pltpu.VMEM)] * 8,
        out_specs=pl.BlockSpec(memory_space=pltpu.SMEM),
    )(adj.reshape(_NW * _K, _S), alp.reshape(_NW * _K, _S), cnt,
      adj0.reshape(_NW * _K, _S), alp0.reshape(_NW * _K, _S), cnt0,
      node_state, node_state0)
    return out[0, 0]


# speedup vs baseline: 2.1958x; 1.0431x over previous
import functools

import jax
import jax.numpy as jnp
from jax import lax
from jax.experimental import pallas as pl
from jax.experimental.pallas import tpu as pltpu
from jax.experimental.pallas import tpu_sc as plsc

_E = 524288
_K = 20
_S = 2048
_B = 16
_C = 2
_NC = 2
_NS = 16
_NW = _NC * _NS
_EPW = _E // _NW
_ST = 21
_CH = 128
_CS = 137
_NCHUNK = _EPW // _CH
_GRP = _CH // 4

_P = (5.629329963841023e-07, 0.9999574661580921, -0.4992063824052593,
      0.3269723524219558, -0.22283471747775338, 0.13076335879445652,
      -0.052623955162732786, 0.01011890169509671)


def _log1p_poly(z):
    acc = jnp.full((16,), _P[7], jnp.float32)
    for c in (_P[6], _P[5], _P[4], _P[3], _P[2], _P[1], _P[0]):
        acc = acc * z + c
    return acc


def _sc_kernel(th_hbm, al_hbm, lb_hbm, sg_hbm,
               th0_hbm, al0_hbm, lb0_hbm, sg0_hbm,
               adj_out, alp_out, cnt_out, adj0_out, alp0_out, cnt0_out,
               th_v0, th_v1, al_v0, al_v1, lb_v0, lb_v1, sg_v0, sg_v1,
               adj_acc, alp_acc, cnt_acc, row_buf, sem0, sem1):
    th_vs, al_vs = (th_v0, th_v1), (al_v0, al_v1)
    lb_vs, sg_vs = (lb_v0, lb_v1), (sg_v0, sg_v1)
    wid = lax.axis_index("s") * _NC + lax.axis_index("c")

    iota = lax.iota(jnp.int32, 16)
    kvs = []
    evs = []
    for j in range(5):
        fl = iota + 16 * j
        kvs.append(lax.rem(fl, _K))
        ev = ((fl >= _K).astype(jnp.int32) + (fl >= 2 * _K).astype(jnp.int32)
              + (fl >= 3 * _K).astype(jnp.int32))
        evs.append(ev)
    ones = jnp.full((16,), 1.0, jnp.float32)
    zeros = jnp.zeros((16,), jnp.float32)

    cmask = [kvs[j] == 0 for j in range(4)]
    i25 = iota * _ST

    def run_branch(th, al, lb, sg, a_out, l_out, c_out):
        @plsc.parallel_loop(0, _ST * _S // 16)
        def zero_kS(i):
            adj_acc[pl.ds(i * 16, 16)] = zeros
            alp_acc[pl.ds(i * 16, 16)] = zeros

        @plsc.parallel_loop(0, _S // 16)
        def zero_S(i):
            cnt_acc[pl.ds(i * 16, 16)] = zeros

        sems = (sem0, sem1)

        def start(ci, b):
            eoff = wid * _EPW + ci * _CH
            pltpu.async_copy(th.at[:, pl.ds(eoff, _CH)],
                             th_vs[b].at[:, pl.ds(0, _CH)], sems[b])
            pltpu.async_copy(al.at[:, pl.ds(eoff, _CH)],
                             al_vs[b].at[:, pl.ds(0, _CH)], sems[b])
            pltpu.async_copy(lb.at[pl.ds(eoff, _CH)], lb_vs[b], sems[b])
            pltpu.async_copy(sg.at[pl.ds(eoff, _CH)], sg_vs[b], sems[b])

        def wait_slot(b):
            pltpu.make_async_copy(th.at[:, pl.ds(0, _CH)],
                                  th_vs[b].at[:, pl.ds(0, _CH)],
                                  sems[b]).wait()
            pltpu.make_async_copy(al.at[:, pl.ds(0, _CH)],
                                  al_vs[b].at[:, pl.ds(0, _CH)],
                                  sems[b]).wait()
            pltpu.make_async_copy(lb.at[pl.ds(0, _CH)], lb_vs[b],
                                  sems[b]).wait()
            pltpu.make_async_copy(sg.at[pl.ds(0, _CH)], sg_vs[b],
                                  sems[b]).wait()

        def compute(b):
            @plsc.parallel_loop(0, _GRP, unroll=2)
            def group_body(g):
                eb = g * 4
                for j in range(5):
                    ev = evs[j] + eb
                    seg = plsc.load_gather(sg_vs[b], [ev])
                    y = plsc.load_gather(lb_vs[b], [ev])
                    tgt = seg * _ST + kvs[j]
                    t = plsc.load_gather(th_vs[b], [kvs[j], ev])
                    z = jnp.exp(-jnp.abs(t))
                    loss = jnp.maximum(t, 0.0) - t * y + _log1p_poly(z)
                    plsc.addupdate_scatter(adj_acc, [tgt], loss)
                    av = plsc.load_gather(al_vs[b], [kvs[j], ev])
                    plsc.addupdate_scatter(alp_acc, [tgt], av)
                    if j < 4:
                        plsc.addupdate_scatter(cnt_acc, [seg], ones,
                                               mask=cmask[j])

        start(0, 0)
        start(1, 1)

        def chunk_pair(i, carry):
            ci = i * 2
            for b in range(2):
                wait_slot(b)
                compute(b)

                @pl.when(ci + 2 + b < _NCHUNK)
                def _():
                    start(ci + 2 + b, b)
            return carry
        lax.fori_loop(0, _NCHUNK // 2, chunk_pair, 0)

        def flush(acc, out2d):
            def row(k, carry):
                @plsc.parallel_loop(0, _S // 16)
                def blk(sb):
                    v = plsc.load_gather(acc, [i25 + (sb * 16 * _ST + k)])
                    row_buf[pl.ds(sb * 16, 16)] = v
                pltpu.sync_copy(row_buf, out2d.at[wid, pl.ds(k * _S, _S)])
                return carry
            lax.fori_loop(0, _K, row, 0)
        flush(adj_acc, a_out)
        flush(alp_acc, l_out)
        pltpu.sync_copy(cnt_acc, c_out.at[wid])

    run_branch(th_hbm, al_hbm, lb_hbm, sg_hbm, adj_out, alp_out, cnt_out)
    run_branch(th0_hbm, al0_hbm, lb0_hbm, sg0_hbm, adj0_out, alp0_out, cnt0_out)


_sc_call = functools.partial(
    pl.kernel,
    out_type=[
        jax.ShapeDtypeStruct((_NW, _K * _S), jnp.float32),
        jax.ShapeDtypeStruct((_NW, _K * _S), jnp.float32),
        jax.ShapeDtypeStruct((_NW, _S), jnp.float32),
        jax.ShapeDtypeStruct((_NW, _K * _S), jnp.float32),
        jax.ShapeDtypeStruct((_NW, _K * _S), jnp.float32),
        jax.ShapeDtypeStruct((_NW, _S), jnp.float32),
    ],
    mesh=plsc.VectorSubcoreMesh(core_axis_name="c", subcore_axis_name="s",
                                num_cores=_NC, num_subcores=_NS),
    scratch_types=[
        pltpu.VMEM((_K, _CS), jnp.float32),
        pltpu.VMEM((_K, _CS), jnp.float32),
        pltpu.VMEM((_K, _CS), jnp.float32),
        pltpu.VMEM((_K, _CS), jnp.float32),
        pltpu.VMEM((_CH,), jnp.float32),
        pltpu.VMEM((_CH,), jnp.float32),
        pltpu.VMEM((_CH,), jnp.int32),
        pltpu.VMEM((_CH,), jnp.int32),
        pltpu.VMEM((_ST * _S,), jnp.float32),
        pltpu.VMEM((_ST * _S,), jnp.float32),
        pltpu.VMEM((_S,), jnp.float32),
        pltpu.VMEM((_S,), jnp.float32),
        pltpu.SemaphoreType.DMA,
        pltpu.SemaphoreType.DMA,
    ],
    compiler_params=pltpu.CompilerParams(needs_layout_passes=False),
)(_sc_kernel)


def _lse0(x):
    m = jnp.max(x, axis=0, keepdims=True)
    return jnp.log(jnp.sum(jnp.exp(x - m), axis=0, keepdims=True)) + m


def _tc_finish_kernel(adj_ref, alp_ref, cnt_ref, adj0_ref, alp0_ref, cnt0_ref,
                      ns_ref, ns0_ref, out_ref):
    f32 = jnp.float32
    ri = lax.broadcasted_iota(jnp.int32, (_S, _B * _C), 0)
    cj = lax.broadcasted_iota(jnp.int32, (_S, _B * _C), 1)
    bucket = (ri // (_S // (_B * _C)) == cj).astype(f32)
    pi = lax.broadcasted_iota(jnp.int32, (_B * _C, _B), 0)
    pj = lax.broadcasted_iota(jnp.int32, (_B * _C, _B), 1)
    even = (pi == 2 * pj).astype(f32)
    odd = (pi == 2 * pj + 1).astype(f32)

    cnt = jnp.sum(cnt_ref[...], axis=0, keepdims=True)
    cnt0 = jnp.sum(cnt0_ref[...], axis=0, keepdims=True)

    def branch(adj_p, alp_p, cnt_norm, cnt_bc):
        adj = jnp.sum(adj_p.reshape(_NW, _K, _S), axis=0)
        alp = jnp.sum(alp_p.reshape(_NW, _K, _S), axis=0) / cnt_norm
        lsm = alp - _lse0(alp)
        log_prob = _lse0(lsm - adj)
        bc_lp = jnp.dot(log_prob, bucket, precision=lax.Precision.HIGHEST)
        bc_cn = jnp.dot(cnt_bc, bucket, precision=lax.Precision.HIGHEST)
        bc_loss = bc_lp / bc_cn
        e = jnp.dot(bc_loss, even, precision=lax.Precision.HIGHEST)
        o = jnp.dot(bc_loss, odd, precision=lax.Precision.HIGHEST)
        m = jnp.maximum(e, o)
        b_loss = -(jnp.log(jnp.exp(e - m) + jnp.exp(o - m)) + m)
        return jnp.sum(b_loss) / _B

    loss = branch(adj_ref[...], alp_ref[...], cnt, cnt)
    loss0 = branch(adj0_ref[...], alp0_ref[...], cnt, cnt0)

    d = ns_ref[...] - ns0_ref[...]
    trace_d = jnp.sum(jnp.sqrt(jnp.sum(d * d, axis=1)))

    out_ref[0, 0] = 10.0 * (loss + loss0) + trace_d


def kernel(label, label0, log_theta, log_theta0, log_alpha, log_alpha0,
           node_state, node_state0, subgraph_idx, subgraph_idx0,
           subgraph_idx_base, subgraph_idx_base0, num_canonical_order):
    adj, alp, cnt, adj0, alp0, cnt0 = _sc_call(
        log_theta.T, log_alpha.T, label, subgraph_idx,
        log_theta0.T, log_alpha0.T, label0, subgraph_idx0)

    out = pl.pallas_call(
        _tc_finish_kernel,
        out_shape=jax.ShapeDtypeStruct((1, 1), jnp.float32),
        in_specs=[pl.BlockSpec(memory_space=pltpu.VMEM)] * 8,
        out_specs=pl.BlockSpec(memory_space=pltpu.SMEM),
    )(adj.reshape(_NW * _K, _S), alp.reshape(_NW * _K, _S), cnt,
      adj0.reshape(_NW * _K, _S), alp0.reshape(_NW * _K, _S), cnt0,
      node_state, node_state0)
    return out[0, 0]
